# Initial kernel scaffold; baseline (speedup 1.0000x reference)
import jax
import jax.numpy as jnp
from jax.experimental import pallas as pl


def kernel(x, pos, edge_index, W11, W12, g1, b1, W21, W22, g2, b2, W31, W32, g3, b3, Wr1, Wr2, br, Wc1, Wc2, bc, Wo1, Wo2, bo):
    raise NotImplementedError("write your pallas kernel here")


# speedup vs baseline: 1.8538x; 1.8538x over previous
import jax
import jax.numpy as jnp
from jax.experimental import pallas as pl
from jax.experimental.pallas import tpu as pltpu

EPS = 1e-5
F32 = jnp.float32


def _sds(shape):
    return jax.ShapeDtypeStruct(shape, F32)


def _dot(a, b):
    return jnp.dot(a, b, preferred_element_type=F32)


def _pre_body(x_ref, p2_ref, wx_ref, wp_ref, a_ref, b_ref):
    b = _dot(p2_ref[...], wp_ref[...])
    a_ref[...] = _dot(x_ref[...], wx_ref[...]) + b
    b_ref[...] = b


def _h_body(m_ref, b_ref, w_ref, h_ref, s_ref, q_ref):
    h = _dot(m_ref[...] - b_ref[...], w_ref[...])
    h_ref[...] = h
    s_ref[...] = jnp.sum(h, 0, keepdims=True)
    q_ref[...] = jnp.sum(h * h, 0, keepdims=True)


def _h2_body(m2_ref, b2_ref, w2_ref, m3_ref, b3_ref, w3_ref,
             h2_ref, s2_ref, q2_ref, h3_ref, s3_ref, q3_ref):
    _h_body(m2_ref, b2_ref, w2_ref, h2_ref, s2_ref, q2_ref)
    _h_body(m3_ref, b3_ref, w3_ref, h3_ref, s3_ref, q3_ref)


def _x1a23_body(h_ref, sc_ref, sh_ref, p2_ref, w2x_ref, w2p_ref,
                w3x_ref, w3p_ref, a2_ref, b2_ref, a3_ref, b3_ref):
    x1 = jnp.maximum(h_ref[...] * sc_ref[...] + sh_ref[...], 0.0)
    b2 = _dot(p2_ref[...], w2p_ref[...])
    a2_ref[...] = _dot(x1, w2x_ref[...]) + b2
    b2_ref[...] = b2
    b3 = _dot(p2_ref[...], w3p_ref[...])
    a3_ref[...] = _dot(x1, w3x_ref[...]) + b3
    b3_ref[...] = b3


def _x23heads_body(h2_ref, sc2_ref, sh2_ref, h3_ref, sc3_ref, sh3_ref, p2_ref,
                   wrx_ref, wrp_ref, wcx_ref, wcp_ref, wox_ref, wop_ref,
                   ar_ref, br_ref, ac_ref, bc_ref, ao_ref, bo_ref):
    x2 = jnp.maximum(h2_ref[...] * sc2_ref[...] + sh2_ref[...], 0.0)
    x3 = jnp.maximum(h3_ref[...] * sc3_ref[...] + sh3_ref[...], 0.0)
    br = _dot(p2_ref[...], wrp_ref[...])
    ar_ref[...] = _dot(x2, wrx_ref[...]) + br
    br_ref[...] = br
    bc = _dot(p2_ref[...], wcp_ref[...])
    ac_ref[...] = _dot(x3, wcx_ref[...]) + bc
    bc_ref[...] = bc
    bo = _dot(p2_ref[...], wop_ref[...])
    ao_ref[...] = _dot(x3, wox_ref[...]) + bo
    bo_ref[...] = bo


def _heads_body(mr_ref, br_ref, wr_ref, vbr_ref, mc_ref, bc_ref, wc_ref,
                vbc_ref, mo_ref, bo_ref, wo_ref, vbo_ref,
                reg_ref, cls_ref, obj_ref):
    reg_ref[...] = _dot(mr_ref[...] - br_ref[...], wr_ref[...]) + vbr_ref[...]
    cls_ref[...] = _dot(mc_ref[...] - bc_ref[...], wc_ref[...]) + vbc_ref[...]
    obj_ref[...] = _dot(mo_ref[...] - bo_ref[...], wo_ref[...]) + vbo_ref[...]


def _bn_coeffs(s, q, g, b, n):
    mu = s / n
    var = q / n - mu * mu
    scale = g[None, :] / jnp.sqrt(var + EPS)
    shift = b[None, :] - mu * scale
    return scale, shift


def _segmax(a, src, dst, n):
    m = jax.ops.segment_max(a[src], dst, num_segments=n)
    return jnp.maximum(m, a)


def kernel(x, pos, edge_index, W11, W12, g1, b1, W21, W22, g2, b2,
           W31, W32, g3, b3, Wr1, Wr2, br, Wc1, Wc2, bc, Wo1, Wo2, bo):
    n, d = x.shape
    c = Wc2.shape[1]
    pos2 = pos[:, :2]
    src = edge_index[0]
    dst = edge_index[1]
    fn = float(n)

    A1, B1 = pl.pallas_call(
        _pre_body, out_shape=[_sds((n, d)), _sds((n, d))])(
            x, pos2, W11[:d], W11[d:])
    M1 = _segmax(A1, src, dst, n)

    h1, s1, q1 = pl.pallas_call(
        _h_body, out_shape=[_sds((n, d)), _sds((1, d)), _sds((1, d))])(
            M1, B1, W12)
    sc1, sh1 = _bn_coeffs(s1, q1, g1, b1, fn)

    A2, B2, A3, B3 = pl.pallas_call(
        _x1a23_body, out_shape=[_sds((n, d))] * 4)(
            h1, sc1, sh1, pos2, W21[:d], W21[d:], W31[:d], W31[d:])
    M2 = _segmax(A2, src, dst, n)
    M3 = _segmax(A3, src, dst, n)

    h2, s2, q2, h3, s3, q3 = pl.pallas_call(
        _h2_body, out_shape=[_sds((n, d)), _sds((1, d)), _sds((1, d))] * 2)(
            M2, B2, W22, M3, B3, W32)
    sc2, sh2 = _bn_coeffs(s2, q2, g2, b2, fn)
    sc3, sh3 = _bn_coeffs(s3, q3, g3, b3, fn)

    Ar, Br, Ac, Bc, Ao, Bo = pl.pallas_call(
        _x23heads_body, out_shape=[_sds((n, d))] * 6)(
            h2, sc2, sh2, h3, sc3, sh3, pos2,
            Wr1[:d], Wr1[d:], Wc1[:d], Wc1[d:], Wo1[:d], Wo1[d:])
    Mr = _segmax(Ar, src, dst, n)
    Mc = _segmax(Ac, src, dst, n)
    Mo = _segmax(Ao, src, dst, n)

    reg, cls, obj = pl.pallas_call(
        _heads_body,
        out_shape=[_sds((n, 4)), _sds((n, c)), _sds((n, 1))])(
            Mr, Br, Wr2, br[None, :], Mc, Bc, Wc2, bc[None, :],
            Mo, Bo, Wo2, bo[None, :])
    return (cls, reg, obj)
